# Initial kernel scaffold; baseline (speedup 1.0000x reference)
import jax
import jax.numpy as jnp
from jax.experimental import pallas as pl


def kernel(x, pe, gamma, beta):
    raise NotImplementedError("write your pallas kernel here")


# speedup vs baseline: 2.3560x; 2.3560x over previous
import jax
import jax.numpy as jnp
from jax.experimental import pallas as pl

EPS = 1e-5
BLK = 512


def _ln_kernel(x_ref, pe_ref, g_ref, b_ref, o_ref):
    h = x_ref[0] + pe_ref[...]
    mean = jnp.mean(h, axis=-1, keepdims=True)
    d = h - mean
    var = jnp.mean(d * d, axis=-1, keepdims=True)
    o_ref[0] = d * jax.lax.rsqrt(var + EPS) * g_ref[...] + b_ref[...]


def kernel(x, pe, gamma, beta):
    B, S, H = x.shape
    g2 = gamma.reshape(1, H)
    b2 = beta.reshape(1, H)
    grid = (S // BLK, B)
    return pl.pallas_call(
        _ln_kernel,
        grid=grid,
        in_specs=[
            pl.BlockSpec((1, BLK, H), lambda s, b: (b, s, 0)),
            pl.BlockSpec((BLK, H), lambda s, b: (s, 0)),
            pl.BlockSpec((1, H), lambda s, b: (0, 0)),
            pl.BlockSpec((1, H), lambda s, b: (0, 0)),
        ],
        out_specs=pl.BlockSpec((1, BLK, H), lambda s, b: (b, s, 0)),
        out_shape=jax.ShapeDtypeStruct((B, S, H), x.dtype),
    )(x, pe, g2, b2)


# speedup vs baseline: 2.6584x; 1.1284x over previous
import jax
import jax.numpy as jnp
from jax.experimental import pallas as pl

EPS = 1e-5
BLK = 1024


def _ln_kernel(x_ref, pe_ref, g_ref, b_ref, o_ref):
    h = x_ref[0] + pe_ref[...]
    mean = jnp.mean(h, axis=-1, keepdims=True)
    d = h - mean
    var = jnp.mean(d * d, axis=-1, keepdims=True)
    o_ref[0] = d * jax.lax.rsqrt(var + EPS) * g_ref[...] + b_ref[...]


def kernel(x, pe, gamma, beta):
    B, S, H = x.shape
    g2 = gamma.reshape(1, H)
    b2 = beta.reshape(1, H)
    grid = (S // BLK, B)
    return pl.pallas_call(
        _ln_kernel,
        grid=grid,
        in_specs=[
            pl.BlockSpec((1, BLK, H), lambda s, b: (b, s, 0)),
            pl.BlockSpec((BLK, H), lambda s, b: (s, 0)),
            pl.BlockSpec((1, H), lambda s, b: (0, 0)),
            pl.BlockSpec((1, H), lambda s, b: (0, 0)),
        ],
        out_specs=pl.BlockSpec((1, BLK, H), lambda s, b: (b, s, 0)),
        out_shape=jax.ShapeDtypeStruct((B, S, H), x.dtype),
    )(x, pe, g2, b2)


# speedup vs baseline: 2.7724x; 1.0429x over previous
import jax
import jax.numpy as jnp
from jax.experimental import pallas as pl

EPS = 1e-5
BLK = 2048


def _ln_kernel(x_ref, pe_ref, g_ref, b_ref, o_ref):
    h = x_ref[0] + pe_ref[...]
    mean = jnp.mean(h, axis=-1, keepdims=True)
    d = h - mean
    var = jnp.mean(d * d, axis=-1, keepdims=True)
    o_ref[0] = d * jax.lax.rsqrt(var + EPS) * g_ref[...] + b_ref[...]


def kernel(x, pe, gamma, beta):
    B, S, H = x.shape
    g2 = gamma.reshape(1, H)
    b2 = beta.reshape(1, H)
    grid = (S // BLK, B)
    return pl.pallas_call(
        _ln_kernel,
        grid=grid,
        in_specs=[
            pl.BlockSpec((1, BLK, H), lambda s, b: (b, s, 0)),
            pl.BlockSpec((BLK, H), lambda s, b: (s, 0)),
            pl.BlockSpec((1, H), lambda s, b: (0, 0)),
            pl.BlockSpec((1, H), lambda s, b: (0, 0)),
        ],
        out_specs=pl.BlockSpec((1, BLK, H), lambda s, b: (b, s, 0)),
        out_shape=jax.ShapeDtypeStruct((B, S, H), x.dtype),
    )(x, pe, g2, b2)
